# baseline (speedup 1.0000x reference)
import jax
import jax.numpy as jnp
from jax.experimental import pallas as pl
from jax.experimental.pallas import tpu as pltpu

_SIM_THRESHOLD = 0.2
_ALPHA = 0.5
_K = 10
_U = 2048
_I = 16384


def _nmat_body(t_ref, out_ref, r_ref):
    k = pl.program_id(0)

    @pl.when(k == 0)
    def _():
        out_ref[...] = jnp.zeros_like(out_ref)

    x = t_ref[...]
    out_ref[...] += jax.lax.dot_general(
        x, x, (((1,), (1,)), ((), ())), preferred_element_type=jnp.float32)

    @pl.when(k == pl.num_programs(0) - 1)
    def _():
        B = 256
        for c in range(_U // B):
            rows = c * B + jax.lax.broadcasted_iota(jnp.int32, (B, _U), 0)
            cols = jax.lax.broadcasted_iota(jnp.int32, (B, _U), 1)
            eye = (rows == cols).astype(jnp.float32)
            r_ref[pl.ds(c * B, B), :] = jnp.sum(
                out_ref[pl.ds(c * B, B), :] * eye, axis=1, keepdims=True)


def _nmat(Tb):
    BK = 2048
    return pl.pallas_call(
        _nmat_body,
        grid=(_I // BK,),
        in_specs=[pl.BlockSpec((_U, BK), lambda k: (0, k))],
        out_specs=[
            pl.BlockSpec((_U, _U), lambda k: (0, 0)),
            pl.BlockSpec((_U, 1), lambda k: (0, 0)),
        ],
        out_shape=[
            jax.ShapeDtypeStruct((_U, _U), jnp.float32),
            jax.ShapeDtypeStruct((_U, 1), jnp.float32),
        ],
        compiler_params=pltpu.CompilerParams(
            dimension_semantics=("arbitrary",)),
    )(Tb)


def _wsel_body(n_ref, rc_ref, rr_ref, w_ref, s_ref):
    B = n_ref.shape[0]
    i = pl.program_id(0)
    N = n_ref[...]
    denom = rc_ref[...] + rr_ref[...] - N
    denom = jnp.where(denom == 0.0, 1.0, denom)
    J = N / denom
    rows = i * B + jax.lax.broadcasted_iota(jnp.int32, (B, _U), 0)
    cols = jax.lax.broadcasted_iota(jnp.int32, (B, _U), 1)
    J = jnp.where(rows == cols, 0.0, J)
    maskf = (J > _SIM_THRESHOLD).astype(jnp.float32)
    counts = jnp.sum(maskf, axis=1, keepdims=True)
    Jc = J
    tk = jnp.zeros_like(J)
    for _ in range(_K):
        m = jnp.max(Jc, axis=1, keepdims=True)
        pos = jnp.min(jnp.where(Jc == m, cols, 2 * _U), axis=1, keepdims=True)
        oh = cols == pos
        tk = jnp.where(oh, 1.0, tk)
        Jc = jnp.where(oh, -1.0, Jc)
    W = jnp.where(counts >= float(_K), maskf, tk)
    w_ref[...] = W.astype(jnp.bfloat16)
    s_ref[...] = jnp.maximum(jnp.sum(W, axis=1, keepdims=True), 1.0)


def _wsel(N, r_col, r_row):
    B = 256
    return pl.pallas_call(
        _wsel_body,
        grid=(_U // B,),
        in_specs=[
            pl.BlockSpec((B, _U), lambda i: (i, 0)),
            pl.BlockSpec((B, 1), lambda i: (i, 0)),
            pl.BlockSpec((1, _U), lambda i: (0, 0)),
        ],
        out_specs=[
            pl.BlockSpec((B, _U), lambda i: (i, 0)),
            pl.BlockSpec((B, 1), lambda i: (i, 0)),
        ],
        out_shape=[
            jax.ShapeDtypeStruct((_U, _U), jnp.bfloat16),
            jax.ShapeDtypeStruct((_U, 1), jnp.float32),
        ],
    )(N, r_col, r_row)


def _dmat_body(w_ref, t_ref, ti_ref, s_ref, d_ref, n_ref):
    k = pl.program_id(1)
    nk = pl.num_programs(1)
    M = jax.lax.dot_general(
        w_ref[...], t_ref[...], (((1,), (0,)), ((), ())),
        preferred_element_type=jnp.float32)
    nm = M / s_ref[...]
    a = jnp.float32(_ALPHA)
    c = jnp.float32(1.0 - _ALPHA)
    D = a * ti_ref[...].astype(jnp.float32) + c * nm
    d_ref[...] = D.astype(jnp.bfloat16)

    @pl.when(k == 0)
    def _():
        n_ref[...] = jnp.zeros_like(n_ref)

    n_ref[...] += jnp.sum(D * D, axis=1, keepdims=True)

    @pl.when(k == nk - 1)
    def _():
        n_ref[...] = jnp.maximum(jnp.sqrt(n_ref[...]), 1e-12)


def _dmat(Wb, Tb, s_col):
    BI, BK = 1024, 2048
    return pl.pallas_call(
        _dmat_body,
        grid=(_U // BI, _I // BK),
        in_specs=[
            pl.BlockSpec((BI, _U), lambda i, k: (i, 0)),
            pl.BlockSpec((_U, BK), lambda i, k: (0, k)),
            pl.BlockSpec((BI, BK), lambda i, k: (i, k)),
            pl.BlockSpec((BI, 1), lambda i, k: (i, 0)),
        ],
        out_specs=[
            pl.BlockSpec((BI, BK), lambda i, k: (i, k)),
            pl.BlockSpec((BI, 1), lambda i, k: (i, 0)),
        ],
        out_shape=[
            jax.ShapeDtypeStruct((_U, _I), jnp.bfloat16),
            jax.ShapeDtypeStruct((_U, 1), jnp.float32),
        ],
        compiler_params=pltpu.CompilerParams(
            dimension_semantics=("parallel", "arbitrary")),
    )(Wb, Tb, Tb, s_col)


def _cos_body(d_ref, dall_ref, nc_ref, nr_ref, vals_ref, idx_ref, g_ref):
    k = pl.program_id(1)
    nk = pl.num_programs(1)

    @pl.when(k == 0)
    def _():
        g_ref[...] = jnp.zeros_like(g_ref)

    g_ref[...] += jax.lax.dot_general(
        d_ref[...], dall_ref[...], (((1,), (1,)), ((), ())),
        preferred_element_type=jnp.float32)

    @pl.when(k == nk - 1)
    def _():
        B = g_ref.shape[0]
        C = (g_ref[...] / nc_ref[...]) / nr_ref[...]
        cols = jax.lax.broadcasted_iota(jnp.int32, (B, _U), 1)
        lane = jax.lax.broadcasted_iota(jnp.int32, (B, _K), 1)
        vals = jnp.zeros((B, _K), jnp.float32)
        idxs = jnp.zeros((B, _K), jnp.int32)
        for t in range(_K):
            m = jnp.max(C, axis=1, keepdims=True)
            pos = jnp.min(jnp.where(C == m, cols, 2 * _U), axis=1,
                          keepdims=True)
            oh = cols == pos
            vals = jnp.where(lane == t, m, vals)
            idxs = jnp.where(lane == t, pos, idxs)
            C = jnp.where(oh, -1.0, C)
        vals_ref[...] = vals
        idx_ref[...] = idxs


def _cos_topk(Dbf, n_col, n_row):
    BI, BK = 1024, 2048
    return pl.pallas_call(
        _cos_body,
        grid=(_U // BI, _I // BK),
        in_specs=[
            pl.BlockSpec((BI, BK), lambda i, k: (i, k)),
            pl.BlockSpec((_U, BK), lambda i, k: (0, k)),
            pl.BlockSpec((BI, 1), lambda i, k: (i, 0)),
            pl.BlockSpec((1, _U), lambda i, k: (0, 0)),
        ],
        out_specs=[
            pl.BlockSpec((BI, _K), lambda i, k: (i, 0)),
            pl.BlockSpec((BI, _K), lambda i, k: (i, 0)),
        ],
        out_shape=[
            jax.ShapeDtypeStruct((_U, _K), jnp.float32),
            jax.ShapeDtypeStruct((_U, _K), jnp.int32),
        ],
        scratch_shapes=[pltpu.VMEM((BI, _U), jnp.float32)],
        compiler_params=pltpu.CompilerParams(
            dimension_semantics=("parallel", "arbitrary")),
    )(Dbf, Dbf, n_col, n_row)


def kernel(train_mat):
    Tb = train_mat.astype(jnp.bfloat16)
    N, r_col = _nmat(Tb)
    r_row = r_col.reshape(1, _U)
    Wb, s_col = _wsel(N, r_col, r_row)
    Dbf, n_col = _dmat(Wb, Tb, s_col)
    n_row = n_col.reshape(1, _U)
    return _cos_topk(Dbf, n_col, n_row)


# speedup vs baseline: 1.1015x; 1.1015x over previous
import jax
import jax.numpy as jnp
from jax.experimental import pallas as pl
from jax.experimental.pallas import tpu as pltpu

_SIM_THRESHOLD = 0.2
_ALPHA = 0.5
_K = 10
_U = 2048
_I = 16384


def _nmat_body(t_ref, out_ref, tb_ref, r_ref):
    k = pl.program_id(0)

    @pl.when(k == 0)
    def _():
        out_ref[...] = jnp.zeros_like(out_ref)

    x = t_ref[...].astype(jnp.bfloat16)
    tb_ref[...] = x
    out_ref[...] += jax.lax.dot_general(
        x, x, (((1,), (1,)), ((), ())), preferred_element_type=jnp.float32)

    @pl.when(k == pl.num_programs(0) - 1)
    def _():
        B = 256
        for c in range(_U // B):
            rows = c * B + jax.lax.broadcasted_iota(jnp.int32, (B, _U), 0)
            cols = jax.lax.broadcasted_iota(jnp.int32, (B, _U), 1)
            eye = (rows == cols).astype(jnp.float32)
            r_ref[pl.ds(c * B, B), :] = jnp.sum(
                out_ref[pl.ds(c * B, B), :] * eye, axis=1, keepdims=True)


def _nmat(T):
    BK = 1024
    return pl.pallas_call(
        _nmat_body,
        grid=(_I // BK,),
        in_specs=[pl.BlockSpec((_U, BK), lambda k: (0, k))],
        out_specs=[
            pl.BlockSpec((_U, _U), lambda k: (0, 0)),
            pl.BlockSpec((_U, BK), lambda k: (0, k)),
            pl.BlockSpec((_U, 1), lambda k: (0, 0)),
        ],
        out_shape=[
            jax.ShapeDtypeStruct((_U, _U), jnp.float32),
            jax.ShapeDtypeStruct((_U, _I), jnp.bfloat16),
            jax.ShapeDtypeStruct((_U, 1), jnp.float32),
        ],
        compiler_params=pltpu.CompilerParams(
            dimension_semantics=("arbitrary",)),
    )(T)


def _dmat_body(n_ref, rc_ref, rr_ref, t_ref, ti_ref, d_ref, nrm_ref,
               w_ref, s_ref):
    i = pl.program_id(0)
    k = pl.program_id(1)
    nk = pl.num_programs(1)

    @pl.when(k == 0)
    def _():
        B = n_ref.shape[0]
        N = n_ref[...]
        denom = rc_ref[...] + rr_ref[...] - N
        denom = jnp.where(denom == 0.0, 1.0, denom)
        J = N / denom
        rows = i * B + jax.lax.broadcasted_iota(jnp.int32, (B, _U), 0)
        cols = jax.lax.broadcasted_iota(jnp.int32, (B, _U), 1)
        J = jnp.where(rows == cols, 0.0, J)
        maskf = (J > _SIM_THRESHOLD).astype(jnp.float32)
        counts = jnp.sum(maskf, axis=1, keepdims=True)
        Jc = J
        tk = jnp.zeros_like(J)
        for _ in range(_K):
            m = jnp.max(Jc, axis=1, keepdims=True)
            pos = jnp.min(jnp.where(Jc == m, cols, 2 * _U), axis=1,
                          keepdims=True)
            oh = cols == pos
            tk = jnp.where(oh, 1.0, tk)
            Jc = jnp.where(oh, -1.0, Jc)
        W = jnp.where(counts >= float(_K), maskf, tk)
        w_ref[...] = W.astype(jnp.bfloat16)
        s_ref[...] = jnp.maximum(jnp.sum(W, axis=1, keepdims=True), 1.0)
        nrm_ref[...] = jnp.zeros_like(nrm_ref)

    M = jax.lax.dot_general(
        w_ref[...], t_ref[...], (((1,), (0,)), ((), ())),
        preferred_element_type=jnp.float32)
    nm = M / s_ref[...]
    a = jnp.float32(_ALPHA)
    c = jnp.float32(1.0 - _ALPHA)
    D = a * ti_ref[...].astype(jnp.float32) + c * nm
    d_ref[...] = D.astype(jnp.bfloat16)
    nrm_ref[...] += jnp.sum(D * D, axis=1, keepdims=True)

    @pl.when(k == nk - 1)
    def _():
        nrm_ref[...] = jnp.maximum(jnp.sqrt(nrm_ref[...]), 1e-12)


def _dmat(N, r_col, r_row, Tb):
    BI, BK = 512, 2048
    return pl.pallas_call(
        _dmat_body,
        grid=(_U // BI, _I // BK),
        in_specs=[
            pl.BlockSpec((BI, _U), lambda i, k: (i, 0)),
            pl.BlockSpec((BI, 1), lambda i, k: (i, 0)),
            pl.BlockSpec((1, _U), lambda i, k: (0, 0)),
            pl.BlockSpec((_U, BK), lambda i, k: (0, k)),
            pl.BlockSpec((BI, BK), lambda i, k: (i, k)),
        ],
        out_specs=[
            pl.BlockSpec((BI, BK), lambda i, k: (i, k)),
            pl.BlockSpec((BI, 1), lambda i, k: (i, 0)),
        ],
        out_shape=[
            jax.ShapeDtypeStruct((_U, _I), jnp.bfloat16),
            jax.ShapeDtypeStruct((_U, 1), jnp.float32),
        ],
        scratch_shapes=[
            pltpu.VMEM((BI, _U), jnp.bfloat16),
            pltpu.VMEM((BI, 1), jnp.float32),
        ],
        compiler_params=pltpu.CompilerParams(
            dimension_semantics=("parallel", "arbitrary")),
    )(N, r_col, r_row, Tb, Tb)


def _cos_body(d_ref, dall_ref, nc_ref, nr_ref, vals_ref, idx_ref, g_ref):
    k = pl.program_id(1)
    nk = pl.num_programs(1)

    @pl.when(k == 0)
    def _():
        g_ref[...] = jnp.zeros_like(g_ref)

    g_ref[...] += jax.lax.dot_general(
        d_ref[...], dall_ref[...], (((1,), (1,)), ((), ())),
        preferred_element_type=jnp.float32)

    @pl.when(k == nk - 1)
    def _():
        B = g_ref.shape[0]
        C = (g_ref[...] / nc_ref[...]) / nr_ref[...]
        cols = jax.lax.broadcasted_iota(jnp.int32, (B, _U), 1)
        lane = jax.lax.broadcasted_iota(jnp.int32, (B, _K), 1)
        vals = jnp.zeros((B, _K), jnp.float32)
        idxs = jnp.zeros((B, _K), jnp.int32)
        for t in range(_K):
            m = jnp.max(C, axis=1, keepdims=True)
            pos = jnp.min(jnp.where(C == m, cols, 2 * _U), axis=1,
                          keepdims=True)
            oh = cols == pos
            vals = jnp.where(lane == t, m, vals)
            idxs = jnp.where(lane == t, pos, idxs)
            C = jnp.where(oh, -1.0, C)
        vals_ref[...] = vals
        idx_ref[...] = idxs


def _cos_topk(Dbf, n_col, n_row):
    BI, BK = 1024, 2048
    return pl.pallas_call(
        _cos_body,
        grid=(_U // BI, _I // BK),
        in_specs=[
            pl.BlockSpec((BI, BK), lambda i, k: (i, k)),
            pl.BlockSpec((_U, BK), lambda i, k: (0, k)),
            pl.BlockSpec((BI, 1), lambda i, k: (i, 0)),
            pl.BlockSpec((1, _U), lambda i, k: (0, 0)),
        ],
        out_specs=[
            pl.BlockSpec((BI, _K), lambda i, k: (i, 0)),
            pl.BlockSpec((BI, _K), lambda i, k: (i, 0)),
        ],
        out_shape=[
            jax.ShapeDtypeStruct((_U, _K), jnp.float32),
            jax.ShapeDtypeStruct((_U, _K), jnp.int32),
        ],
        scratch_shapes=[pltpu.VMEM((BI, _U), jnp.float32)],
        compiler_params=pltpu.CompilerParams(
            dimension_semantics=("parallel", "arbitrary")),
    )(Dbf, Dbf, n_col, n_row)


def kernel(train_mat):
    N, Tb, r_col = _nmat(train_mat)
    r_row = r_col.reshape(1, _U)
    Dbf, n_col = _dmat(N, r_col, r_row, Tb)
    n_row = n_col.reshape(1, _U)
    return _cos_topk(Dbf, n_col, n_row)
